# Initial kernel scaffold; baseline (speedup 1.0000x reference)
import jax
import jax.numpy as jnp
from jax.experimental import pallas as pl


def kernel(x, edge_index, batch, rootindex, W1, b1, W2, b2):
    raise NotImplementedError("write your pallas kernel here")


# speedup vs baseline: 11.0924x; 11.0924x over previous
import functools

import jax
import jax.numpy as jnp
from jax import lax
from jax.experimental import pallas as pl
from jax.experimental.pallas import tpu as pltpu
from jax.experimental.pallas import tpu_sc as plsc

N = 10000
E = 320000
B = 128
IN_F = 128
HID_F = 64

NP = 10240
NC = 2
NS = 16
NW = NC * NS
EP = 327680
CW = 128
CHUNKS = EP // (NW * CW)
ROWS_PER_TILE = NP // NS
RB = 256
NBLK = NP // RB


def _deg_body(dst_hbm, out_hbm, idx_d, ones_v, sem, acc):
    cid = lax.axis_index("c")
    sid = lax.axis_index("s")
    wid = cid * NS + sid
    pltpu.sync_copy(dst_hbm.at[pl.ds(wid * CHUNKS, CHUNKS)], idx_d)

    @pl.loop(0, CW)
    def _fill(r):
        ones_v[r, :] = jnp.full((16,), 1.0, jnp.float32)

    @pl.loop(0, CW)
    def _zero(r):
        ones_v[r, :] = jnp.zeros((16,), jnp.float32)

    @pl.loop(0, ROWS_PER_TILE // CW)
    def _zcopy(k):
        pltpu.sync_copy(ones_v, acc.at[pl.ds(sid * ROWS_PER_TILE + k * CW, CW)])

    @pl.loop(0, CW)
    def _refill(r):
        ones_v[r, :] = jnp.full((16,), 1.0, jnp.float32)

    plsc.subcore_barrier()

    @pl.loop(0, CHUNKS)
    def _scatter(j):
        pltpu.sync_copy(ones_v, acc.at[idx_d.at[j]], add=True)

    plsc.subcore_barrier()
    pltpu.sync_copy(acc.at[pl.ds(sid * ROWS_PER_TILE, ROWS_PER_TILE)],
                    out_hbm.at[cid, pl.ds(sid * ROWS_PER_TILE, ROWS_PER_TILE)])


@functools.cache
def _get_sc_deg():
    mesh = plsc.VectorSubcoreMesh(core_axis_name="c", subcore_axis_name="s")
    return pl.kernel(
        _deg_body,
        out_type=jax.ShapeDtypeStruct((NC, NP, 16), jnp.float32),
        mesh=mesh,
        scratch_types=[
            pltpu.VMEM((CHUNKS, CW), jnp.int32),
            pltpu.VMEM((CW, 16), jnp.float32),
            pltpu.SemaphoreType.DMA,
            pltpu.VMEM_SHARED((NP, 16), jnp.float32),
        ],
    )


def _edge_body(u_hbm, src_hbm, dst_hbm, out_hbm, idx_s, idx_d, rows, zbuf, gsem, acc):
    cid = lax.axis_index("c")
    sid = lax.axis_index("s")
    wid = cid * NS + sid
    pltpu.sync_copy(src_hbm.at[pl.ds(wid * CHUNKS, CHUNKS)], idx_s)
    pltpu.sync_copy(dst_hbm.at[pl.ds(wid * CHUNKS, CHUNKS)], idx_d)

    @pl.loop(0, CW)
    def _zero(r):
        for c in range(4):
            zbuf[r, pl.ds(c * 16, 16)] = jnp.zeros((16,), jnp.float32)

    @pl.loop(0, ROWS_PER_TILE // CW)
    def _zcopy(k):
        pltpu.sync_copy(zbuf, acc.at[pl.ds(sid * ROWS_PER_TILE + k * CW, CW)])

    plsc.subcore_barrier()

    @pl.loop(0, CHUNKS)
    def _chunk(j):
        pltpu.async_copy(u_hbm.at[idx_s.at[j]], rows, gsem).wait()
        pltpu.sync_copy(rows, acc.at[idx_d.at[j]], add=True)

    plsc.subcore_barrier()
    pltpu.sync_copy(acc.at[pl.ds(sid * ROWS_PER_TILE, ROWS_PER_TILE)],
                    out_hbm.at[cid, pl.ds(sid * ROWS_PER_TILE, ROWS_PER_TILE)])


@functools.cache
def _get_sc_edge():
    mesh = plsc.VectorSubcoreMesh(core_axis_name="c", subcore_axis_name="s")
    return pl.kernel(
        _edge_body,
        out_type=jax.ShapeDtypeStruct((NC, NP, HID_F), jnp.float32),
        mesh=mesh,
        compiler_params=pltpu.CompilerParams(use_tc_tiling_on_sc=False),
        scratch_types=[
            pltpu.VMEM((CHUNKS, CW), jnp.int32),
            pltpu.VMEM((CHUNKS, CW), jnp.int32),
            pltpu.VMEM((CW, HID_F), jnp.float32),
            pltpu.VMEM((CW, HID_F), jnp.float32),
            pltpu.SemaphoreType.DMA,
            pltpu.VMEM_SHARED((NP, HID_F), jnp.float32),
        ],
    )


_HI = jax.lax.Precision.HIGHEST


def _dinv_from(deg_ref):
    deg = deg_ref[0, :, 0] + deg_ref[1, :, 0] + 1.0
    return lax.rsqrt(deg)


def _tc1_body(x_ref, w1_ref, deg_ref, ridx_ref, w2b_ref, u1_ref, rootsw_ref):
    i = pl.program_id(0)
    dinv = _dinv_from(deg_ref)
    h = jnp.dot(x_ref[...], w1_ref[...], precision=_HI,
                preferred_element_type=jnp.float32)
    u1_ref[...] = h * dinv[:, None]
    rows = i * RB + lax.broadcasted_iota(jnp.int32, (RB, 1), 0)
    sel = (ridx_ref[:, 0][None, :] == rows).astype(jnp.float32)
    part = lax.dot_general(sel, x_ref[...], (((0,), (0,)), ((), ())),
                           precision=_HI, preferred_element_type=jnp.float32)
    contrib = jnp.dot(jnp.maximum(part, 0.0), w2b_ref[...], precision=_HI,
                      preferred_element_type=jnp.float32)

    @pl.when(i == 0)
    def _():
        rootsw_ref[...] = jnp.zeros_like(rootsw_ref)

    rootsw_ref[...] += contrib


def _tc2_body(s1_ref, u1_ref, deg_ref, b1_ref, rootsw_ref, batch_ref, w2a_ref,
              x2_ref, u2_ref):
    dinv = _dinv_from(deg_ref)
    x2 = (s1_ref[0] + s1_ref[1] + u1_ref[...]) * dinv[:, None] + b1_ref[0:1, :]
    x2_ref[...] = x2
    hr = jnp.maximum(x2, 0.0)
    cols = lax.broadcasted_iota(jnp.int32, (1, B), 1)
    bsel = (batch_ref[:, 0:1] == cols).astype(jnp.float32)
    rext = jnp.dot(bsel, rootsw_ref[...], precision=_HI,
                   preferred_element_type=jnp.float32)
    u2_ref[...] = (jnp.dot(hr, w2a_ref[...], precision=_HI,
                           preferred_element_type=jnp.float32) + rext) * dinv[:, None]


def _tc3_body(s2_ref, u2_ref, deg_ref, b2_ref, x2_ref, batch_ref, ridx_ref,
              out_ref, seg_ref, root_ref, cnt_ref):
    i = pl.program_id(0)

    @pl.when(i == 0)
    def _():
        seg_ref[...] = jnp.zeros_like(seg_ref)
        root_ref[...] = jnp.zeros_like(root_ref)
        cnt_ref[...] = jnp.zeros_like(cnt_ref)

    dinv = _dinv_from(deg_ref)
    g = jnp.maximum((s2_ref[0] + s2_ref[1] + u2_ref[...]) * dinv[:, None]
                    + b2_ref[0:1, :], 0.0)
    cols = lax.broadcasted_iota(jnp.int32, (1, B), 1)
    bsel = (batch_ref[:, 0:1] == cols).astype(jnp.float32)
    seg_ref[...] += lax.dot_general(bsel, g, (((0,), (0,)), ((), ())),
                                    precision=_HI, preferred_element_type=jnp.float32)
    ones = jnp.ones((RB, HID_F), jnp.float32)
    cnt_ref[...] += lax.dot_general(bsel, ones, (((0,), (0,)), ((), ())),
                                    precision=_HI, preferred_element_type=jnp.float32)
    rows = i * RB + lax.broadcasted_iota(jnp.int32, (1, RB), 1)
    rsel = (ridx_ref[:, 0][:, None] == rows).astype(jnp.float32)
    root_ref[...] += jnp.dot(rsel, x2_ref[...], precision=_HI,
                             preferred_element_type=jnp.float32)

    @pl.when(i == NBLK - 1)
    def _():
        cnt = cnt_ref[...]
        first = seg_ref[...] / jnp.maximum(cnt, 1.0)
        second = jnp.where(cnt > 0, root_ref[...], 0.0)
        out_ref[...] = jnp.concatenate([first, second], axis=1)


def _row_spec(shape):
    return pl.BlockSpec(shape, lambda i: (i, 0))


def _fix_spec(shape):
    return pl.BlockSpec(shape, lambda i: (0, 0))


_DEG_SPEC = pl.BlockSpec((NC, RB, 16), lambda i: (0, i, 0))
_PAIR_SPEC = pl.BlockSpec((NC, RB, HID_F), lambda i: (0, i, 0))

_tc1 = pl.pallas_call(
    _tc1_body,
    grid=(NBLK,),
    in_specs=[
        _row_spec((RB, IN_F)),
        _fix_spec((IN_F, HID_F)),
        _DEG_SPEC,
        _fix_spec((B, 1)),
        _fix_spec((IN_F, HID_F)),
    ],
    out_specs=[_row_spec((RB, HID_F)), _fix_spec((B, HID_F))],
    out_shape=[
        jax.ShapeDtypeStruct((NP, HID_F), jnp.float32),
        jax.ShapeDtypeStruct((B, HID_F), jnp.float32),
    ],
)

_tc2 = pl.pallas_call(
    _tc2_body,
    grid=(NBLK,),
    in_specs=[
        _PAIR_SPEC,
        _row_spec((RB, HID_F)),
        _DEG_SPEC,
        _fix_spec((8, HID_F)),
        _fix_spec((B, HID_F)),
        _row_spec((RB, 1)),
        _fix_spec((HID_F, HID_F)),
    ],
    out_specs=[_row_spec((RB, HID_F)), _row_spec((RB, HID_F))],
    out_shape=[
        jax.ShapeDtypeStruct((NP, HID_F), jnp.float32),
        jax.ShapeDtypeStruct((NP, HID_F), jnp.float32),
    ],
)

_tc3 = pl.pallas_call(
    _tc3_body,
    grid=(NBLK,),
    in_specs=[
        _PAIR_SPEC,
        _row_spec((RB, HID_F)),
        _DEG_SPEC,
        _fix_spec((8, HID_F)),
        _row_spec((RB, HID_F)),
        _row_spec((RB, 1)),
        _fix_spec((B, 1)),
    ],
    out_specs=pl.BlockSpec((B, B), lambda i: (0, 0)),
    out_shape=jax.ShapeDtypeStruct((B, B), jnp.float32),
    scratch_shapes=[
        pltpu.VMEM((B, HID_F), jnp.float32),
        pltpu.VMEM((B, HID_F), jnp.float32),
        pltpu.VMEM((B, HID_F), jnp.float32),
    ],
)


@jax.jit
def kernel(x, edge_index, batch, rootindex, W1, b1, W2, b2):
    xp = jnp.pad(x, ((0, NP - N), (0, 0)))
    src = jnp.concatenate([edge_index[0], jnp.zeros((EP - E,), jnp.int32)])
    dst = jnp.concatenate([edge_index[1], jnp.full((EP - E,), N, jnp.int32)])
    src2d = src.reshape(NW * CHUNKS, CW)
    dst2d = dst.reshape(NW * CHUNKS, CW)
    batchp = jnp.concatenate([batch, jnp.full((NP - N,), B, jnp.int32)])
    batch2d = batchp.reshape(NP, 1)
    ridx2d = rootindex.reshape(B, 1)
    b1t = jnp.tile(b1.reshape(1, HID_F), (8, 1))
    b2t = jnp.tile(b2.reshape(1, HID_F), (8, 1))
    w2a = W2[:HID_F]
    w2b = W2[HID_F:]

    sc_deg = _get_sc_deg()
    sc_edge = _get_sc_edge()
    deg2 = sc_deg(dst2d)
    u1, rootsw = _tc1(xp, W1, deg2, ridx2d, w2b)
    s1 = sc_edge(u1, src2d, dst2d)
    x2, u2 = _tc2(s1, u1, deg2, b1t, rootsw, batch2d, w2a)
    s2 = sc_edge(u2, src2d, dst2d)
    out = _tc3(s2, u2, deg2, b2t, x2, batch2d, ridx2d)
    return out


# speedup vs baseline: 28.6952x; 2.5869x over previous
import functools

import jax
import jax.numpy as jnp
from jax import lax
from jax.experimental import pallas as pl
from jax.experimental.pallas import tpu as pltpu
from jax.experimental.pallas import tpu_sc as plsc

N = 10000
E = 320000
B = 128
IN_F = 128
HID_F = 64

NP = 10240
NC = 2
NS = 16
NW = NC * NS
EP = 327680
CW = 128
CHUNKS = EP // (NW * CW)
ROWS_PER_TILE = NP // NS
RB = 256
NBLK = NP // RB


def _deg_body(dst_hbm, out_hbm, idx_d, ones_v, sem, acc):
    cid = lax.axis_index("c")
    sid = lax.axis_index("s")
    wid = cid * NS + sid
    pltpu.sync_copy(dst_hbm.at[pl.ds(wid * CHUNKS, CHUNKS)], idx_d)

    @pl.loop(0, CW)
    def _zero(r):
        ones_v[r, :] = jnp.zeros((16,), jnp.float32)

    @pl.loop(0, ROWS_PER_TILE // CW)
    def _zcopy(k):
        pltpu.sync_copy(ones_v, acc.at[pl.ds(sid * ROWS_PER_TILE + k * CW, CW)])

    @pl.loop(0, CW)
    def _refill(r):
        ones_v[r, :] = jnp.full((16,), 1.0, jnp.float32)

    plsc.subcore_barrier()

    @pl.loop(0, CHUNKS)
    def _scatter(j):
        pltpu.sync_copy(ones_v, acc.at[idx_d.at[j]], add=True)

    plsc.subcore_barrier()
    pltpu.sync_copy(acc.at[pl.ds(sid * ROWS_PER_TILE, ROWS_PER_TILE)],
                    out_hbm.at[cid, pl.ds(sid * ROWS_PER_TILE, ROWS_PER_TILE)])


@functools.cache
def _get_sc_deg():
    mesh = plsc.VectorSubcoreMesh(core_axis_name="c", subcore_axis_name="s")
    return pl.kernel(
        _deg_body,
        out_type=jax.ShapeDtypeStruct((NC, NP, 16), jnp.float32),
        mesh=mesh,
        scratch_types=[
            pltpu.VMEM((CHUNKS, CW), jnp.int32),
            pltpu.VMEM((CW, 16), jnp.float32),
            pltpu.SemaphoreType.DMA,
            pltpu.VMEM_SHARED((NP, 16), jnp.float32),
        ],
    )


def _edge_body(u_hbm, src_hbm, dst_hbm, out_hbm, idx_s, idx_d, rows, zbuf,
               gsem, ustage, acc):
    cid = lax.axis_index("c")
    sid = lax.axis_index("s")
    wid = cid * NS + sid
    pltpu.sync_copy(src_hbm.at[pl.ds(wid * CHUNKS, CHUNKS)], idx_s)
    pltpu.sync_copy(dst_hbm.at[pl.ds(wid * CHUNKS, CHUNKS)], idx_d)

    pltpu.sync_copy(u_hbm.at[pl.ds(sid * ROWS_PER_TILE, ROWS_PER_TILE)],
                    ustage.at[pl.ds(sid * ROWS_PER_TILE, ROWS_PER_TILE)])

    @pl.loop(0, CW)
    def _zero(r):
        for c in range(HID_F // 16):
            zbuf[r, pl.ds(c * 16, 16)] = jnp.zeros((16,), jnp.float32)

    @pl.loop(0, ROWS_PER_TILE // CW)
    def _zcopy(k):
        pltpu.sync_copy(zbuf, acc.at[pl.ds(sid * ROWS_PER_TILE + k * CW, CW)])

    plsc.subcore_barrier()

    pltpu.async_copy(ustage.at[idx_s.at[0]], rows.at[0], gsem)

    @pl.loop(0, CHUNKS)
    def _chunk(j):
        b = jnp.bitwise_and(j, 1)
        pltpu.make_async_copy(ustage.at[idx_s.at[j]], rows.at[b], gsem).wait()

        @pl.when(j < CHUNKS - 1)
        def _():
            pltpu.async_copy(ustage.at[idx_s.at[j + 1]], rows.at[1 - b], gsem)

        pltpu.sync_copy(rows.at[b], acc.at[idx_d.at[j]], add=True)

    plsc.subcore_barrier()
    pltpu.sync_copy(acc.at[pl.ds(sid * ROWS_PER_TILE, ROWS_PER_TILE)],
                    out_hbm.at[cid, pl.ds(sid * ROWS_PER_TILE, ROWS_PER_TILE)])


@functools.cache
def _get_sc_edge():
    mesh = plsc.VectorSubcoreMesh(core_axis_name="c", subcore_axis_name="s")
    return pl.kernel(
        _edge_body,
        out_type=jax.ShapeDtypeStruct((NC, NP, HID_F), jnp.float32),
        mesh=mesh,
        compiler_params=pltpu.CompilerParams(use_tc_tiling_on_sc=False),
        scratch_types=[
            pltpu.VMEM((CHUNKS, CW), jnp.int32),
            pltpu.VMEM((CHUNKS, CW), jnp.int32),
            pltpu.VMEM((2, CW, HID_F), jnp.float32),
            pltpu.VMEM((CW, HID_F), jnp.float32),
            pltpu.SemaphoreType.DMA,
            pltpu.VMEM_SHARED((NP, HID_F), jnp.float32),
            pltpu.VMEM_SHARED((NP, HID_F), jnp.float32),
        ],
    )


def _dinv_from(deg_ref):
    deg = deg_ref[0, :, 0] + deg_ref[1, :, 0] + 1.0
    return lax.rsqrt(deg)


def _tc1a_body(x_ref, w1_ref, ridx_ref, w2b_ref, h1_ref, rootsw_ref):
    i = pl.program_id(0)
    h1_ref[...] = jnp.dot(x_ref[...], w1_ref[...])
    rows = i * RB + lax.broadcasted_iota(jnp.int32, (1, RB), 1)
    rsel = (ridx_ref[...] == rows).astype(jnp.float32)
    part = jnp.dot(rsel, x_ref[...])
    contrib = jnp.dot(jnp.maximum(part, 0.0), w2b_ref[...])

    @pl.when(i == 0)
    def _():
        rootsw_ref[...] = jnp.zeros_like(rootsw_ref)

    rootsw_ref[...] += contrib


def _tc1b_body(h1_ref, deg_ref, u1_ref):
    dinv = _dinv_from(deg_ref)
    u1_ref[...] = h1_ref[...] * dinv[:, None]


def _tc2_body(s1_ref, u1_ref, deg_ref, b1_ref, rootsw_ref, batch_ref, w2a_ref,
              x2_ref, u2_ref):
    dinv = _dinv_from(deg_ref)
    x2 = (s1_ref[0] + s1_ref[1] + u1_ref[...]) * dinv[:, None] + b1_ref[0:1, :]
    x2_ref[...] = x2
    hr = jnp.maximum(x2, 0.0)
    cols = lax.broadcasted_iota(jnp.int32, (1, B), 1)
    bsel = (batch_ref[:, 0:1] == cols).astype(jnp.float32)
    rext = jnp.dot(bsel, rootsw_ref[...])
    u2_ref[...] = (jnp.dot(hr, w2a_ref[...]) + rext) * dinv[:, None]


def _tc3_body(s2_ref, u2_ref, deg_ref, b2_ref, x2_ref, batchT_ref, ridx_ref,
              out_ref, seg_ref, root_ref, cnt_ref):
    i = pl.program_id(0)

    @pl.when(i == 0)
    def _():
        seg_ref[...] = jnp.zeros_like(seg_ref)
        root_ref[...] = jnp.zeros_like(root_ref)
        cnt_ref[...] = jnp.zeros_like(cnt_ref)

    dinv = _dinv_from(deg_ref)
    g = jnp.maximum((s2_ref[0] + s2_ref[1] + u2_ref[...]) * dinv[:, None]
                    + b2_ref[0:1, :], 0.0)
    biota = lax.broadcasted_iota(jnp.int32, (B, 1), 0)
    bselT = (batchT_ref[0] == biota).astype(jnp.float32)
    seg_ref[...] += jnp.dot(bselT, g)
    ones = jnp.ones((RB, HID_F), jnp.float32)
    cnt_ref[...] += jnp.dot(bselT, ones)
    rows = i * RB + lax.broadcasted_iota(jnp.int32, (1, RB), 1)
    rsel = (ridx_ref[...] == rows).astype(jnp.float32)
    root_ref[...] += jnp.dot(rsel, x2_ref[...])

    @pl.when(i == NBLK - 1)
    def _():
        cnt = cnt_ref[...]
        first = seg_ref[...] / jnp.maximum(cnt, 1.0)
        second = jnp.where(cnt > 0, root_ref[...], 0.0)
        out_ref[...] = jnp.concatenate([first, second], axis=1)


def _row_spec(shape):
    return pl.BlockSpec(shape, lambda i: (i, 0))


def _fix_spec(shape):
    return pl.BlockSpec(shape, lambda i: (0, 0))


_DEG_SPEC = pl.BlockSpec((NC, RB, 16), lambda i: (0, i, 0))
_PAIR_SPEC = pl.BlockSpec((NC, RB, HID_F), lambda i: (0, i, 0))

_tc1a = pl.pallas_call(
    _tc1a_body,
    grid=(NBLK,),
    in_specs=[
        _row_spec((RB, IN_F)),
        _fix_spec((IN_F, HID_F)),
        _fix_spec((B, 1)),
        _fix_spec((IN_F, HID_F)),
    ],
    out_specs=[_row_spec((RB, HID_F)), _fix_spec((B, HID_F))],
    out_shape=[
        jax.ShapeDtypeStruct((NP, HID_F), jnp.float32),
        jax.ShapeDtypeStruct((B, HID_F), jnp.float32),
    ],
)

_tc1b = pl.pallas_call(
    _tc1b_body,
    grid=(NBLK,),
    in_specs=[_row_spec((RB, HID_F)), _DEG_SPEC],
    out_specs=_row_spec((RB, HID_F)),
    out_shape=jax.ShapeDtypeStruct((NP, HID_F), jnp.float32),
)

_tc2 = pl.pallas_call(
    _tc2_body,
    grid=(NBLK,),
    in_specs=[
        _PAIR_SPEC,
        _row_spec((RB, HID_F)),
        _DEG_SPEC,
        _fix_spec((8, HID_F)),
        _fix_spec((B, HID_F)),
        _row_spec((RB, 1)),
        _fix_spec((HID_F, HID_F)),
    ],
    out_specs=[_row_spec((RB, HID_F)), _row_spec((RB, HID_F))],
    out_shape=[
        jax.ShapeDtypeStruct((NP, HID_F), jnp.float32),
        jax.ShapeDtypeStruct((NP, HID_F), jnp.float32),
    ],
)

_tc3 = pl.pallas_call(
    _tc3_body,
    grid=(NBLK,),
    in_specs=[
        _PAIR_SPEC,
        _row_spec((RB, HID_F)),
        _DEG_SPEC,
        _fix_spec((8, HID_F)),
        _row_spec((RB, HID_F)),
        pl.BlockSpec((1, 1, RB), lambda i: (i, 0, 0)),
        _fix_spec((B, 1)),
    ],
    out_specs=pl.BlockSpec((B, B), lambda i: (0, 0)),
    out_shape=jax.ShapeDtypeStruct((B, B), jnp.float32),
    scratch_shapes=[
        pltpu.VMEM((B, HID_F), jnp.float32),
        pltpu.VMEM((B, HID_F), jnp.float32),
        pltpu.VMEM((B, HID_F), jnp.float32),
    ],
)


@jax.jit
def kernel(x, edge_index, batch, rootindex, W1, b1, W2, b2):
    xp = jnp.pad(x, ((0, NP - N), (0, 0)))
    src = jnp.concatenate([edge_index[0], jnp.zeros((EP - E,), jnp.int32)])
    dst = jnp.concatenate([edge_index[1], jnp.full((EP - E,), N, jnp.int32)])
    src2d = src.reshape(NW * CHUNKS, CW)
    dst2d = dst.reshape(NW * CHUNKS, CW)
    batchp = jnp.concatenate([batch, jnp.full((NP - N,), B, jnp.int32)])
    batch2d = batchp.reshape(NP, 1)
    batchT = batchp.reshape(NBLK, 1, RB)
    ridx2d = rootindex.reshape(B, 1)
    b1t = jnp.tile(b1.reshape(1, HID_F), (8, 1))
    b2t = jnp.tile(b2.reshape(1, HID_F), (8, 1))
    w2a = W2[:HID_F]
    w2b = W2[HID_F:]

    sc_deg = _get_sc_deg()
    sc_edge = _get_sc_edge()
    deg2 = sc_deg(dst2d)
    h1, rootsw = _tc1a(xp, W1, ridx2d, w2b)
    u1 = _tc1b(h1, deg2)
    s1 = sc_edge(u1, src2d, dst2d)
    x2, u2 = _tc2(s1, u1, deg2, b1t, rootsw, batch2d, w2a)
    s2 = sc_edge(u2, src2d, dst2d)
    out = _tc3(s2, u2, deg2, b2t, x2, batchT, ridx2d)
    return out


# speedup vs baseline: 32.6866x; 1.1391x over previous
import functools

import jax
import jax.numpy as jnp
from jax import lax
from jax.experimental import pallas as pl
from jax.experimental.pallas import tpu as pltpu
from jax.experimental.pallas import tpu_sc as plsc

N = 10000
E = 320000
B = 128
IN_F = 128
HID_F = 64

NP = 10240
NC = 2
NS = 16
NW = NC * NS
EP = 327680
CW = 128
CHUNKS = EP // (NW * CW)
ROWS_PER_TILE = NP // NS
RB = 512
NBLK = NP // RB


def _deg_body(dst_hbm, out_hbm, idx_d, ones_v, sem, acc):
    cid = lax.axis_index("c")
    sid = lax.axis_index("s")
    wid = cid * NS + sid
    pltpu.sync_copy(dst_hbm.at[pl.ds(wid * CHUNKS, CHUNKS)], idx_d)

    @pl.loop(0, CW)
    def _zero(r):
        ones_v[r, :] = jnp.zeros((16,), jnp.float32)

    @pl.loop(0, ROWS_PER_TILE // CW)
    def _zcopy(k):
        pltpu.sync_copy(ones_v, acc.at[pl.ds(sid * ROWS_PER_TILE + k * CW, CW)])

    @pl.loop(0, CW)
    def _refill(r):
        ones_v[r, :] = jnp.full((16,), 1.0, jnp.float32)

    plsc.subcore_barrier()

    @pl.loop(0, CHUNKS)
    def _scatter(j):
        pltpu.sync_copy(ones_v, acc.at[idx_d.at[j]], add=True)

    plsc.subcore_barrier()
    pltpu.sync_copy(acc.at[pl.ds(sid * ROWS_PER_TILE, ROWS_PER_TILE)],
                    out_hbm.at[cid, pl.ds(sid * ROWS_PER_TILE, ROWS_PER_TILE)])


@functools.cache
def _get_sc_deg():
    mesh = plsc.VectorSubcoreMesh(core_axis_name="c", subcore_axis_name="s")
    return pl.kernel(
        _deg_body,
        out_type=jax.ShapeDtypeStruct((NC, NP, 16), jnp.float32),
        mesh=mesh,
        scratch_types=[
            pltpu.VMEM((CHUNKS, CW), jnp.int32),
            pltpu.VMEM((CW, 16), jnp.float32),
            pltpu.SemaphoreType.DMA,
            pltpu.VMEM_SHARED((NP, 16), jnp.float32),
        ],
    )


def _edge_body(u_hbm, src_hbm, dst_hbm, out_hbm, idx_s, idx_d, rows, zbuf,
               gsem, ssem, ustage, acc):
    cid = lax.axis_index("c")
    sid = lax.axis_index("s")
    wid = cid * NS + sid
    pltpu.sync_copy(src_hbm.at[pl.ds(wid * CHUNKS, CHUNKS)], idx_s)
    pltpu.sync_copy(dst_hbm.at[pl.ds(wid * CHUNKS, CHUNKS)], idx_d)

    pltpu.sync_copy(u_hbm.at[pl.ds(sid * ROWS_PER_TILE, ROWS_PER_TILE)],
                    ustage.at[pl.ds(sid * ROWS_PER_TILE, ROWS_PER_TILE)])

    @pl.loop(0, CW)
    def _zero(r):
        for c in range(HID_F // 16):
            zbuf[r, pl.ds(c * 16, 16)] = jnp.zeros((16,), jnp.float32)

    @pl.loop(0, ROWS_PER_TILE // CW)
    def _zcopy(k):
        pltpu.sync_copy(zbuf, acc.at[pl.ds(sid * ROWS_PER_TILE + k * CW, CW)])

    plsc.subcore_barrier()

    pltpu.async_copy(ustage.at[idx_s.at[0]], rows.at[0], gsem)

    @pl.loop(0, CHUNKS)
    def _chunk(j):
        b = jnp.bitwise_and(j, 1)
        pltpu.make_async_copy(ustage.at[idx_s.at[j]], rows.at[b], gsem).wait()

        @pl.when(j > 0)
        def _():
            pltpu.make_async_copy(rows.at[1 - b], acc.at[idx_d.at[j - 1]],
                                  ssem).wait()

        @pl.when(j < CHUNKS - 1)
        def _():
            pltpu.async_copy(ustage.at[idx_s.at[j + 1]], rows.at[1 - b], gsem)

        pltpu.async_copy(rows.at[b], acc.at[idx_d.at[j]], ssem, add=True)

    _LB = (CHUNKS - 1) & 1
    pltpu.make_async_copy(rows.at[_LB], acc.at[idx_d.at[CHUNKS - 1]],
                          ssem).wait()

    plsc.subcore_barrier()
    pltpu.sync_copy(acc.at[pl.ds(sid * ROWS_PER_TILE, ROWS_PER_TILE)],
                    out_hbm.at[cid, pl.ds(sid * ROWS_PER_TILE, ROWS_PER_TILE)])


@functools.cache
def _get_sc_edge():
    mesh = plsc.VectorSubcoreMesh(core_axis_name="c", subcore_axis_name="s")
    return pl.kernel(
        _edge_body,
        out_type=jax.ShapeDtypeStruct((NC, NP, HID_F), jnp.float32),
        mesh=mesh,
        compiler_params=pltpu.CompilerParams(use_tc_tiling_on_sc=False),
        scratch_types=[
            pltpu.VMEM((CHUNKS, CW), jnp.int32),
            pltpu.VMEM((CHUNKS, CW), jnp.int32),
            pltpu.VMEM((2, CW, HID_F), jnp.float32),
            pltpu.VMEM((CW, HID_F), jnp.float32),
            pltpu.SemaphoreType.DMA,
            pltpu.SemaphoreType.DMA,
            pltpu.VMEM_SHARED((NP, HID_F), jnp.float32),
            pltpu.VMEM_SHARED((NP, HID_F), jnp.float32),
        ],
    )


def _dinv_from(deg_ref):
    deg = deg_ref[0, :, 0] + deg_ref[1, :, 0] + 1.0
    return lax.rsqrt(deg)


def _tc1a_body(x_ref, w1_ref, ridx_ref, w2b_ref, h1_ref, rootsw_ref):
    i = pl.program_id(0)
    h1_ref[...] = jnp.dot(x_ref[...], w1_ref[...])
    rows = i * RB + lax.broadcasted_iota(jnp.int32, (1, RB), 1)
    rsel = (ridx_ref[...] == rows).astype(jnp.float32)
    part = jnp.dot(rsel, x_ref[...])
    contrib = jnp.dot(jnp.maximum(part, 0.0), w2b_ref[...])

    @pl.when(i == 0)
    def _():
        rootsw_ref[...] = jnp.zeros_like(rootsw_ref)

    rootsw_ref[...] += contrib


def _tc1b_body(h1_ref, deg_ref, u1_ref):
    dinv = _dinv_from(deg_ref)
    u1_ref[...] = h1_ref[...] * dinv[:, None]


def _tc2_body(s1_ref, u1_ref, deg_ref, b1_ref, rootsw_ref, batch_ref, w2a_ref,
              x2_ref, u2_ref):
    dinv = _dinv_from(deg_ref)
    x2 = (s1_ref[0] + s1_ref[1] + u1_ref[...]) * dinv[:, None] + b1_ref[0:1, :]
    x2_ref[...] = x2
    hr = jnp.maximum(x2, 0.0)
    cols = lax.broadcasted_iota(jnp.int32, (1, B), 1)
    bsel = (batch_ref[:, 0:1] == cols).astype(jnp.float32)
    rext = jnp.dot(bsel, rootsw_ref[...])
    u2_ref[...] = (jnp.dot(hr, w2a_ref[...]) + rext) * dinv[:, None]


def _tc3_body(s2_ref, u2_ref, deg_ref, b2_ref, x2_ref, batchT_ref, ridx_ref,
              out_ref, seg_ref, root_ref, cnt_ref):
    i = pl.program_id(0)

    @pl.when(i == 0)
    def _():
        seg_ref[...] = jnp.zeros_like(seg_ref)
        root_ref[...] = jnp.zeros_like(root_ref)
        cnt_ref[...] = jnp.zeros_like(cnt_ref)

    dinv = _dinv_from(deg_ref)
    g = jnp.maximum((s2_ref[0] + s2_ref[1] + u2_ref[...]) * dinv[:, None]
                    + b2_ref[0:1, :], 0.0)
    biota = lax.broadcasted_iota(jnp.int32, (B, 1), 0)
    bselT = (batchT_ref[0] == biota).astype(jnp.float32)
    seg_ref[...] += jnp.dot(bselT, g)
    ones = jnp.ones((RB, HID_F), jnp.float32)
    cnt_ref[...] += jnp.dot(bselT, ones)
    rows = i * RB + lax.broadcasted_iota(jnp.int32, (1, RB), 1)
    rsel = (ridx_ref[...] == rows).astype(jnp.float32)
    root_ref[...] += jnp.dot(rsel, x2_ref[...])

    @pl.when(i == NBLK - 1)
    def _():
        cnt = cnt_ref[...]
        first = seg_ref[...] / jnp.maximum(cnt, 1.0)
        second = jnp.where(cnt > 0, root_ref[...], 0.0)
        out_ref[...] = jnp.concatenate([first, second], axis=1)


def _row_spec(shape):
    return pl.BlockSpec(shape, lambda i: (i, 0))


def _fix_spec(shape):
    return pl.BlockSpec(shape, lambda i: (0, 0))


_DEG_SPEC = pl.BlockSpec((NC, RB, 16), lambda i: (0, i, 0))
_PAIR_SPEC = pl.BlockSpec((NC, RB, HID_F), lambda i: (0, i, 0))

_tc1a = pl.pallas_call(
    _tc1a_body,
    grid=(NBLK,),
    in_specs=[
        _row_spec((RB, IN_F)),
        _fix_spec((IN_F, HID_F)),
        _fix_spec((B, 1)),
        _fix_spec((IN_F, HID_F)),
    ],
    out_specs=[_row_spec((RB, HID_F)), _fix_spec((B, HID_F))],
    out_shape=[
        jax.ShapeDtypeStruct((NP, HID_F), jnp.float32),
        jax.ShapeDtypeStruct((B, HID_F), jnp.float32),
    ],
)

_tc1b = pl.pallas_call(
    _tc1b_body,
    grid=(NBLK,),
    in_specs=[_row_spec((RB, HID_F)), _DEG_SPEC],
    out_specs=_row_spec((RB, HID_F)),
    out_shape=jax.ShapeDtypeStruct((NP, HID_F), jnp.float32),
)

_tc2 = pl.pallas_call(
    _tc2_body,
    grid=(NBLK,),
    in_specs=[
        _PAIR_SPEC,
        _row_spec((RB, HID_F)),
        _DEG_SPEC,
        _fix_spec((8, HID_F)),
        _fix_spec((B, HID_F)),
        _row_spec((RB, 1)),
        _fix_spec((HID_F, HID_F)),
    ],
    out_specs=[_row_spec((RB, HID_F)), _row_spec((RB, HID_F))],
    out_shape=[
        jax.ShapeDtypeStruct((NP, HID_F), jnp.float32),
        jax.ShapeDtypeStruct((NP, HID_F), jnp.float32),
    ],
)

_tc3 = pl.pallas_call(
    _tc3_body,
    grid=(NBLK,),
    in_specs=[
        _PAIR_SPEC,
        _row_spec((RB, HID_F)),
        _DEG_SPEC,
        _fix_spec((8, HID_F)),
        _row_spec((RB, HID_F)),
        pl.BlockSpec((1, 1, RB), lambda i: (i, 0, 0)),
        _fix_spec((B, 1)),
    ],
    out_specs=pl.BlockSpec((B, B), lambda i: (0, 0)),
    out_shape=jax.ShapeDtypeStruct((B, B), jnp.float32),
    scratch_shapes=[
        pltpu.VMEM((B, HID_F), jnp.float32),
        pltpu.VMEM((B, HID_F), jnp.float32),
        pltpu.VMEM((B, HID_F), jnp.float32),
    ],
)


@jax.jit
def kernel(x, edge_index, batch, rootindex, W1, b1, W2, b2):
    xp = jnp.pad(x, ((0, NP - N), (0, 0)))
    src = jnp.concatenate([edge_index[0], jnp.zeros((EP - E,), jnp.int32)])
    dst = jnp.concatenate([edge_index[1], jnp.full((EP - E,), N, jnp.int32)])
    src2d = src.reshape(NW * CHUNKS, CW)
    dst2d = dst.reshape(NW * CHUNKS, CW)
    batchp = jnp.concatenate([batch, jnp.full((NP - N,), B, jnp.int32)])
    batch2d = batchp.reshape(NP, 1)
    batchT = batchp.reshape(NBLK, 1, RB)
    ridx2d = rootindex.reshape(B, 1)
    b1t = jnp.tile(b1.reshape(1, HID_F), (8, 1))
    b2t = jnp.tile(b2.reshape(1, HID_F), (8, 1))
    w2a = W2[:HID_F]
    w2b = W2[HID_F:]

    sc_deg = _get_sc_deg()
    sc_edge = _get_sc_edge()
    deg2 = sc_deg(dst2d)
    h1, rootsw = _tc1a(xp, W1, ridx2d, w2b)
    u1 = _tc1b(h1, deg2)
    s1 = sc_edge(u1, src2d, dst2d)
    x2, u2 = _tc2(s1, u1, deg2, b1t, rootsw, batch2d, w2a)
    s2 = sc_edge(u2, src2d, dst2d)
    out = _tc3(s2, u2, deg2, b2t, x2, batchT, ridx2d)
    return out


# speedup vs baseline: 34.4693x; 1.0545x over previous
import functools

import jax
import jax.numpy as jnp
from jax import lax
from jax.experimental import pallas as pl
from jax.experimental.pallas import tpu as pltpu
from jax.experimental.pallas import tpu_sc as plsc

N = 10000
E = 320000
B = 128
IN_F = 128
HID_F = 64

NP = 10240
NC = 2
NS = 16
NW = NC * NS
EP = 327680
CW = 128
CHUNKS = EP // (NW * CW)
ROWS_PER_TILE = NP // NS
RB = 1024
NBLK = NP // RB


def _deg_body(dst_hbm, out_hbm, idx_d, ones_v, sem, acc):
    cid = lax.axis_index("c")
    sid = lax.axis_index("s")
    wid = cid * NS + sid
    pltpu.sync_copy(dst_hbm.at[pl.ds(wid * CHUNKS, CHUNKS)], idx_d)

    @pl.loop(0, CW)
    def _zero(r):
        ones_v[r, :] = jnp.zeros((16,), jnp.float32)

    @pl.loop(0, ROWS_PER_TILE // CW)
    def _zcopy(k):
        pltpu.sync_copy(ones_v, acc.at[pl.ds(sid * ROWS_PER_TILE + k * CW, CW)])

    @pl.loop(0, CW)
    def _refill(r):
        ones_v[r, :] = jnp.full((16,), 1.0, jnp.float32)

    plsc.subcore_barrier()

    @pl.loop(0, CHUNKS)
    def _scatter(j):
        pltpu.sync_copy(ones_v, acc.at[idx_d.at[j]], add=True)

    plsc.subcore_barrier()
    pltpu.sync_copy(acc.at[pl.ds(sid * ROWS_PER_TILE, ROWS_PER_TILE)],
                    out_hbm.at[cid, pl.ds(sid * ROWS_PER_TILE, ROWS_PER_TILE)])


@functools.cache
def _get_sc_deg():
    mesh = plsc.VectorSubcoreMesh(core_axis_name="c", subcore_axis_name="s")
    return pl.kernel(
        _deg_body,
        out_type=jax.ShapeDtypeStruct((NC, NP, 16), jnp.float32),
        mesh=mesh,
        scratch_types=[
            pltpu.VMEM((CHUNKS, CW), jnp.int32),
            pltpu.VMEM((CW, 16), jnp.float32),
            pltpu.SemaphoreType.DMA,
            pltpu.VMEM_SHARED((NP, 16), jnp.float32),
        ],
    )


def _edge_body(u_hbm, src_hbm, dst_hbm, out_hbm, idx_s, idx_d, rows, zbuf,
               gsem, ssem, ustage, acc):
    cid = lax.axis_index("c")
    sid = lax.axis_index("s")
    wid = cid * NS + sid
    pltpu.sync_copy(src_hbm.at[pl.ds(wid * CHUNKS, CHUNKS)], idx_s)
    pltpu.sync_copy(dst_hbm.at[pl.ds(wid * CHUNKS, CHUNKS)], idx_d)

    pltpu.sync_copy(u_hbm.at[pl.ds(sid * ROWS_PER_TILE, ROWS_PER_TILE)],
                    ustage.at[pl.ds(sid * ROWS_PER_TILE, ROWS_PER_TILE)])

    @pl.loop(0, CW)
    def _zero(r):
        for c in range(HID_F // 16):
            zbuf[r, pl.ds(c * 16, 16)] = jnp.zeros((16,), jnp.float32)

    @pl.loop(0, ROWS_PER_TILE // CW)
    def _zcopy(k):
        pltpu.sync_copy(zbuf, acc.at[pl.ds(sid * ROWS_PER_TILE + k * CW, CW)])

    plsc.subcore_barrier()

    pltpu.async_copy(ustage.at[idx_s.at[0]], rows.at[0], gsem)

    @pl.loop(0, CHUNKS)
    def _chunk(j):
        b = jnp.bitwise_and(j, 1)
        pltpu.make_async_copy(ustage.at[idx_s.at[j]], rows.at[b], gsem).wait()

        @pl.when(j > 0)
        def _():
            pltpu.make_async_copy(rows.at[1 - b], acc.at[idx_d.at[j - 1]],
                                  ssem).wait()

        @pl.when(j < CHUNKS - 1)
        def _():
            pltpu.async_copy(ustage.at[idx_s.at[j + 1]], rows.at[1 - b], gsem)

        pltpu.async_copy(rows.at[b], acc.at[idx_d.at[j]], ssem, add=True)

    _LB = (CHUNKS - 1) & 1
    pltpu.make_async_copy(rows.at[_LB], acc.at[idx_d.at[CHUNKS - 1]],
                          ssem).wait()

    plsc.subcore_barrier()
    pltpu.sync_copy(acc.at[pl.ds(sid * ROWS_PER_TILE, ROWS_PER_TILE)],
                    out_hbm.at[cid, pl.ds(sid * ROWS_PER_TILE, ROWS_PER_TILE)])


@functools.cache
def _get_sc_edge():
    mesh = plsc.VectorSubcoreMesh(core_axis_name="c", subcore_axis_name="s")
    return pl.kernel(
        _edge_body,
        out_type=jax.ShapeDtypeStruct((NC, NP, HID_F), jnp.float32),
        mesh=mesh,
        compiler_params=pltpu.CompilerParams(use_tc_tiling_on_sc=False),
        scratch_types=[
            pltpu.VMEM((CHUNKS, CW), jnp.int32),
            pltpu.VMEM((CHUNKS, CW), jnp.int32),
            pltpu.VMEM((2, CW, HID_F), jnp.float32),
            pltpu.VMEM((CW, HID_F), jnp.float32),
            pltpu.SemaphoreType.DMA,
            pltpu.SemaphoreType.DMA,
            pltpu.VMEM_SHARED((NP, HID_F), jnp.float32),
            pltpu.VMEM_SHARED((NP, HID_F), jnp.float32),
        ],
    )


def _dinv_from(deg_ref):
    deg = deg_ref[0, :, 0] + deg_ref[1, :, 0] + 1.0
    return lax.rsqrt(deg)


def _tc1a_body(x_ref, w1_ref, ridx_ref, w2b_ref, h1_ref, rootsw_ref):
    i = pl.program_id(0)
    h1_ref[...] = jnp.dot(x_ref[...], w1_ref[...])
    rows = i * RB + lax.broadcasted_iota(jnp.int32, (1, RB), 1)
    rsel = (ridx_ref[...] == rows).astype(jnp.float32)
    part = jnp.dot(rsel, x_ref[...])
    contrib = jnp.dot(jnp.maximum(part, 0.0), w2b_ref[...])

    @pl.when(i == 0)
    def _():
        rootsw_ref[...] = jnp.zeros_like(rootsw_ref)

    rootsw_ref[...] += contrib


def _tc1b_body(h1_ref, deg_ref, u1_ref):
    dinv = _dinv_from(deg_ref)
    u1_ref[...] = h1_ref[...] * dinv[:, None]


def _tc2_body(s1_ref, u1_ref, deg_ref, b1_ref, rootsw_ref, batch_ref, w2a_ref,
              x2_ref, u2_ref):
    dinv = _dinv_from(deg_ref)
    x2 = (s1_ref[0] + s1_ref[1] + u1_ref[...]) * dinv[:, None] + b1_ref[0:1, :]
    x2_ref[...] = x2
    hr = jnp.maximum(x2, 0.0)
    cols = lax.broadcasted_iota(jnp.int32, (1, B), 1)
    bsel = (batch_ref[:, 0:1] == cols).astype(jnp.float32)
    rext = jnp.dot(bsel, rootsw_ref[...])
    u2_ref[...] = (jnp.dot(hr, w2a_ref[...]) + rext) * dinv[:, None]


def _tc3_body(s2_ref, u2_ref, deg_ref, b2_ref, x2_ref, batchT_ref, ridx_ref,
              out_ref, seg_ref, root_ref, cnt_ref):
    i = pl.program_id(0)

    @pl.when(i == 0)
    def _():
        seg_ref[...] = jnp.zeros_like(seg_ref)
        root_ref[...] = jnp.zeros_like(root_ref)
        cnt_ref[...] = jnp.zeros_like(cnt_ref)

    dinv = _dinv_from(deg_ref)
    g = jnp.maximum((s2_ref[0] + s2_ref[1] + u2_ref[...]) * dinv[:, None]
                    + b2_ref[0:1, :], 0.0)
    biota = lax.broadcasted_iota(jnp.int32, (B, 1), 0)
    bselT = (batchT_ref[0] == biota).astype(jnp.float32)
    seg_ref[...] += jnp.dot(bselT, g)
    ones = jnp.ones((RB, HID_F), jnp.float32)
    cnt_ref[...] += jnp.dot(bselT, ones)
    rows = i * RB + lax.broadcasted_iota(jnp.int32, (1, RB), 1)
    rsel = (ridx_ref[...] == rows).astype(jnp.float32)
    root_ref[...] += jnp.dot(rsel, x2_ref[...])

    @pl.when(i == NBLK - 1)
    def _():
        cnt = cnt_ref[...]
        first = seg_ref[...] / jnp.maximum(cnt, 1.0)
        second = jnp.where(cnt > 0, root_ref[...], 0.0)
        out_ref[...] = jnp.concatenate([first, second], axis=1)


def _row_spec(shape):
    return pl.BlockSpec(shape, lambda i: (i, 0))


def _fix_spec(shape):
    return pl.BlockSpec(shape, lambda i: (0, 0))


_DEG_SPEC = pl.BlockSpec((NC, RB, 16), lambda i: (0, i, 0))
_PAIR_SPEC = pl.BlockSpec((NC, RB, HID_F), lambda i: (0, i, 0))
_U_SPEC = pl.BlockSpec((RB, HID_F), lambda i: (i, 0))

_tc1a = pl.pallas_call(
    _tc1a_body,
    grid=(NBLK,),
    in_specs=[
        _row_spec((RB, IN_F)),
        _fix_spec((IN_F, HID_F)),
        _fix_spec((B, 1)),
        _fix_spec((IN_F, HID_F)),
    ],
    out_specs=[_row_spec((RB, HID_F)), _fix_spec((B, HID_F))],
    out_shape=[
        jax.ShapeDtypeStruct((NP, HID_F), jnp.float32),
        jax.ShapeDtypeStruct((B, HID_F), jnp.float32),
    ],
)

_tc1b = pl.pallas_call(
    _tc1b_body,
    grid=(NBLK,),
    in_specs=[_row_spec((RB, HID_F)), _DEG_SPEC],
    out_specs=_U_SPEC,
    out_shape=jax.ShapeDtypeStruct((NP, HID_F), jnp.float32),
)

_tc2 = pl.pallas_call(
    _tc2_body,
    grid=(NBLK,),
    in_specs=[
        _PAIR_SPEC,
        _U_SPEC,
        _DEG_SPEC,
        _fix_spec((8, HID_F)),
        _fix_spec((B, HID_F)),
        _row_spec((RB, 1)),
        _fix_spec((HID_F, HID_F)),
    ],
    out_specs=[_row_spec((RB, HID_F)), _U_SPEC],
    out_shape=[
        jax.ShapeDtypeStruct((NP, HID_F), jnp.float32),
        jax.ShapeDtypeStruct((NP, HID_F), jnp.float32),
    ],
)

_tc3 = pl.pallas_call(
    _tc3_body,
    grid=(NBLK,),
    in_specs=[
        _PAIR_SPEC,
        _U_SPEC,
        _DEG_SPEC,
        _fix_spec((8, HID_F)),
        _row_spec((RB, HID_F)),
        pl.BlockSpec((1, 1, RB), lambda i: (i, 0, 0)),
        _fix_spec((B, 1)),
    ],
    out_specs=pl.BlockSpec((B, B), lambda i: (0, 0)),
    out_shape=jax.ShapeDtypeStruct((B, B), jnp.float32),
    scratch_shapes=[
        pltpu.VMEM((B, HID_F), jnp.float32),
        pltpu.VMEM((B, HID_F), jnp.float32),
        pltpu.VMEM((B, HID_F), jnp.float32),
    ],
)


@jax.jit
def kernel(x, edge_index, batch, rootindex, W1, b1, W2, b2):
    xp = jnp.pad(x, ((0, NP - N), (0, 0)))
    src = jnp.concatenate([edge_index[0], jnp.zeros((EP - E,), jnp.int32)])
    dst = jnp.concatenate([edge_index[1], jnp.full((EP - E,), N, jnp.int32)])
    src2d = src.reshape(NW * CHUNKS, CW)
    dst2d = dst.reshape(NW * CHUNKS, CW)
    batchp = jnp.concatenate([batch, jnp.full((NP - N,), B, jnp.int32)])
    batch2d = batchp.reshape(NP, 1)
    batchT = batchp.reshape(NBLK, 1, RB)
    ridx2d = rootindex.reshape(B, 1)
    b1t = jnp.tile(b1.reshape(1, HID_F), (8, 1))
    b2t = jnp.tile(b2.reshape(1, HID_F), (8, 1))
    w2a = W2[:HID_F]
    w2b = W2[HID_F:]

    sc_deg = _get_sc_deg()
    sc_edge = _get_sc_edge()
    deg2 = sc_deg(dst2d)
    h1, rootsw = _tc1a(xp, W1, ridx2d, w2b)
    u1 = _tc1b(h1, deg2)
    s1 = sc_edge(u1, src2d, dst2d)
    x2, u2 = _tc2(s1, u1, deg2, b1t, rootsw, batch2d, w2a)
    s2 = sc_edge(u2, src2d, dst2d)
    out = _tc3(s2, u2, deg2, b2t, x2, batchT, ridx2d)
    return out


# speedup vs baseline: 36.8671x; 1.0696x over previous
import functools

import jax
import jax.numpy as jnp
from jax import lax
from jax.experimental import pallas as pl
from jax.experimental.pallas import tpu as pltpu
from jax.experimental.pallas import tpu_sc as plsc

N = 10000
E = 320000
B = 128
IN_F = 128
HID_F = 64

NP = 10240
NC = 2
NS = 16
NW = NC * NS
CW = 128
EROWS = E // CW
CHUNKS = EROWS // NW
XTRA = EROWS - NW * CHUNKS
ROWS_PER_TILE = NP // NS
RB = 1024
NBLK = NP // RB


def _deg_body(ei_hbm, out_hbm, idx_d, ones_v, sem, acc):
    cid = lax.axis_index("c")
    sid = lax.axis_index("s")
    wid = cid * NS + sid
    nch = CHUNKS + (wid < XTRA).astype(jnp.int32)
    pltpu.sync_copy(ei_hbm.at[1, pl.ds(wid * CHUNKS, CHUNKS)],
                    idx_d.at[pl.ds(0, CHUNKS)])

    @pl.when(wid < XTRA)
    def _():
        pltpu.sync_copy(ei_hbm.at[1, pl.ds(NW * CHUNKS + wid, 1)],
                        idx_d.at[pl.ds(CHUNKS, 1)])

    @pl.loop(0, CW)
    def _zero(r):
        ones_v[r, :] = jnp.zeros((16,), jnp.float32)

    @pl.loop(0, ROWS_PER_TILE // CW)
    def _zcopy(k):
        pltpu.sync_copy(ones_v, acc.at[pl.ds(sid * ROWS_PER_TILE + k * CW, CW)])

    @pl.loop(0, CW)
    def _refill(r):
        ones_v[r, :] = jnp.full((16,), 1.0, jnp.float32)

    plsc.subcore_barrier()

    @pl.loop(0, nch)
    def _scatter(j):
        pltpu.sync_copy(ones_v, acc.at[idx_d.at[j]], add=True)

    plsc.subcore_barrier()
    pltpu.sync_copy(acc.at[pl.ds(sid * ROWS_PER_TILE, ROWS_PER_TILE)],
                    out_hbm.at[cid, pl.ds(sid * ROWS_PER_TILE, ROWS_PER_TILE)])


@functools.cache
def _get_sc_deg():
    mesh = plsc.VectorSubcoreMesh(core_axis_name="c", subcore_axis_name="s")
    return pl.kernel(
        _deg_body,
        out_type=jax.ShapeDtypeStruct((NC, NP, 16), jnp.float32),
        mesh=mesh,
        compiler_params=pltpu.CompilerParams(use_tc_tiling_on_sc=False),
        scratch_types=[
            pltpu.VMEM((CHUNKS + 1, CW), jnp.int32),
            pltpu.VMEM((CW, 16), jnp.float32),
            pltpu.SemaphoreType.DMA,
            pltpu.VMEM_SHARED((NP, 16), jnp.float32),
        ],
    )


def _edge_body(u_hbm, ei_hbm, out_hbm, idx_s, idx_d, rows, zbuf,
               gsem, ssem, ustage, acc):
    cid = lax.axis_index("c")
    sid = lax.axis_index("s")
    wid = cid * NS + sid
    nch = CHUNKS + (wid < XTRA).astype(jnp.int32)
    pltpu.sync_copy(ei_hbm.at[0, pl.ds(wid * CHUNKS, CHUNKS)],
                    idx_s.at[pl.ds(0, CHUNKS)])
    pltpu.sync_copy(ei_hbm.at[1, pl.ds(wid * CHUNKS, CHUNKS)],
                    idx_d.at[pl.ds(0, CHUNKS)])

    @pl.when(wid < XTRA)
    def _():
        pltpu.sync_copy(ei_hbm.at[0, pl.ds(NW * CHUNKS + wid, 1)],
                        idx_s.at[pl.ds(CHUNKS, 1)])
        pltpu.sync_copy(ei_hbm.at[1, pl.ds(NW * CHUNKS + wid, 1)],
                        idx_d.at[pl.ds(CHUNKS, 1)])

    pltpu.sync_copy(u_hbm.at[pl.ds(sid * ROWS_PER_TILE, ROWS_PER_TILE)],
                    ustage.at[pl.ds(sid * ROWS_PER_TILE, ROWS_PER_TILE)])

    @pl.loop(0, CW)
    def _zero(r):
        for c in range(HID_F // 16):
            zbuf[r, pl.ds(c * 16, 16)] = jnp.zeros((16,), jnp.float32)

    @pl.loop(0, ROWS_PER_TILE // CW)
    def _zcopy(k):
        pltpu.sync_copy(zbuf, acc.at[pl.ds(sid * ROWS_PER_TILE + k * CW, CW)])

    plsc.subcore_barrier()

    pltpu.async_copy(ustage.at[idx_s.at[0]], rows.at[0], gsem)

    @pl.loop(0, nch)
    def _chunk(j):
        b = jnp.bitwise_and(j, 1)
        pltpu.make_async_copy(ustage.at[idx_s.at[j]], rows.at[b], gsem).wait()

        @pl.when(j > 0)
        def _():
            pltpu.make_async_copy(rows.at[1 - b], acc.at[idx_d.at[j - 1]],
                                  ssem).wait()

        @pl.when(j < nch - 1)
        def _():
            pltpu.async_copy(ustage.at[idx_s.at[j + 1]], rows.at[1 - b], gsem)

        pltpu.async_copy(rows.at[b], acc.at[idx_d.at[j]], ssem, add=True)

    _lb = jnp.bitwise_and(nch - 1, 1)
    pltpu.make_async_copy(rows.at[_lb], acc.at[idx_d.at[nch - 1]],
                          ssem).wait()

    plsc.subcore_barrier()
    pltpu.sync_copy(acc.at[pl.ds(sid * ROWS_PER_TILE, ROWS_PER_TILE)],
                    out_hbm.at[cid, pl.ds(sid * ROWS_PER_TILE, ROWS_PER_TILE)])


@functools.cache
def _get_sc_edge():
    mesh = plsc.VectorSubcoreMesh(core_axis_name="c", subcore_axis_name="s")
    return pl.kernel(
        _edge_body,
        out_type=jax.ShapeDtypeStruct((NC, NP, HID_F), jnp.float32),
        mesh=mesh,
        compiler_params=pltpu.CompilerParams(use_tc_tiling_on_sc=False),
        scratch_types=[
            pltpu.VMEM((CHUNKS + 1, CW), jnp.int32),
            pltpu.VMEM((CHUNKS + 1, CW), jnp.int32),
            pltpu.VMEM((2, CW, HID_F), jnp.float32),
            pltpu.VMEM((CW, HID_F), jnp.float32),
            pltpu.SemaphoreType.DMA,
            pltpu.SemaphoreType.DMA,
            pltpu.VMEM_SHARED((NP, HID_F), jnp.float32),
            pltpu.VMEM_SHARED((NP, HID_F), jnp.float32),
        ],
    )


def _dinv_from(deg_ref):
    deg = deg_ref[0, :, 0] + deg_ref[1, :, 0] + 1.0
    return lax.rsqrt(deg)


def _tc1a_body(x_ref, w1_ref, ridx_ref, w2b_ref, h1_ref, rootsw_ref):
    i = pl.program_id(0)
    h1_ref[...] = jnp.dot(x_ref[...], w1_ref[...])
    rows = i * RB + lax.broadcasted_iota(jnp.int32, (1, RB), 1)
    rsel = (ridx_ref[...] == rows).astype(jnp.float32)
    part = jnp.dot(rsel, x_ref[...])
    contrib = jnp.dot(jnp.maximum(part, 0.0), w2b_ref[...])

    @pl.when(i == 0)
    def _():
        rootsw_ref[...] = jnp.zeros_like(rootsw_ref)

    rootsw_ref[...] += contrib


def _tc1b_body(h1_ref, deg_ref, u1_ref):
    dinv = _dinv_from(deg_ref)
    u1_ref[...] = h1_ref[...] * dinv[:, None]


def _tc2_body(s1_ref, u1_ref, deg_ref, b1_ref, rootsw_ref, batch_ref, w2a_ref,
              x2_ref, u2_ref):
    dinv = _dinv_from(deg_ref)
    x2 = (s1_ref[0] + s1_ref[1] + u1_ref[...]) * dinv[:, None] + b1_ref[0:1, :]
    x2_ref[...] = x2
    hr = jnp.maximum(x2, 0.0)
    cols = lax.broadcasted_iota(jnp.int32, (1, B), 1)
    bsel = (batch_ref[:, 0:1] == cols).astype(jnp.float32)
    rext = jnp.dot(bsel, rootsw_ref[...])
    u2_ref[...] = (jnp.dot(hr, w2a_ref[...]) + rext) * dinv[:, None]


def _tc3_body(s2_ref, u2_ref, deg_ref, b2_ref, x2_ref, batchT_ref, ridx_ref,
              out_ref, seg_ref, root_ref, cnt_ref):
    i = pl.program_id(0)

    @pl.when(i == 0)
    def _():
        seg_ref[...] = jnp.zeros_like(seg_ref)
        root_ref[...] = jnp.zeros_like(root_ref)
        cnt_ref[...] = jnp.zeros_like(cnt_ref)

    dinv = _dinv_from(deg_ref)
    g = jnp.maximum((s2_ref[0] + s2_ref[1] + u2_ref[...]) * dinv[:, None]
                    + b2_ref[0:1, :], 0.0)
    biota = lax.broadcasted_iota(jnp.int32, (B, 1), 0)
    bselT = (batchT_ref[0] == biota).astype(jnp.float32)
    seg_ref[...] += jnp.dot(bselT, g)
    ones = jnp.ones((RB, HID_F), jnp.float32)
    cnt_ref[...] += jnp.dot(bselT, ones)
    rows = i * RB + lax.broadcasted_iota(jnp.int32, (1, RB), 1)
    rsel = (ridx_ref[...] == rows).astype(jnp.float32)
    root_ref[...] += jnp.dot(rsel, x2_ref[...])

    @pl.when(i == NBLK - 1)
    def _():
        cnt = cnt_ref[...]
        first = seg_ref[...] / jnp.maximum(cnt, 1.0)
        second = jnp.where(cnt > 0, root_ref[...], 0.0)
        out_ref[...] = jnp.concatenate([first, second], axis=1)


def _row_spec(shape):
    return pl.BlockSpec(shape, lambda i: (i, 0))


def _fix_spec(shape):
    return pl.BlockSpec(shape, lambda i: (0, 0))


_DEG_SPEC = pl.BlockSpec((NC, RB, 16), lambda i: (0, i, 0))
_PAIR_SPEC = pl.BlockSpec((NC, RB, HID_F), lambda i: (0, i, 0))
_U_SPEC = pl.BlockSpec((RB, HID_F), lambda i: (i, 0))

_tc1a = pl.pallas_call(
    _tc1a_body,
    grid=(NBLK,),
    in_specs=[
        _row_spec((RB, IN_F)),
        _fix_spec((IN_F, HID_F)),
        _fix_spec((B, 1)),
        _fix_spec((IN_F, HID_F)),
    ],
    out_specs=[_row_spec((RB, HID_F)), _fix_spec((B, HID_F))],
    out_shape=[
        jax.ShapeDtypeStruct((NP, HID_F), jnp.float32),
        jax.ShapeDtypeStruct((B, HID_F), jnp.float32),
    ],
)

_tc1b = pl.pallas_call(
    _tc1b_body,
    grid=(NBLK,),
    in_specs=[_row_spec((RB, HID_F)), _DEG_SPEC],
    out_specs=_U_SPEC,
    out_shape=jax.ShapeDtypeStruct((NP, HID_F), jnp.float32),
)

_tc2 = pl.pallas_call(
    _tc2_body,
    grid=(NBLK,),
    in_specs=[
        _PAIR_SPEC,
        _U_SPEC,
        _DEG_SPEC,
        _fix_spec((8, HID_F)),
        _fix_spec((B, HID_F)),
        _row_spec((RB, 1)),
        _fix_spec((HID_F, HID_F)),
    ],
    out_specs=[_row_spec((RB, HID_F)), _U_SPEC],
    out_shape=[
        jax.ShapeDtypeStruct((NP, HID_F), jnp.float32),
        jax.ShapeDtypeStruct((NP, HID_F), jnp.float32),
    ],
)

_tc3 = pl.pallas_call(
    _tc3_body,
    grid=(NBLK,),
    in_specs=[
        _PAIR_SPEC,
        _U_SPEC,
        _DEG_SPEC,
        _fix_spec((8, HID_F)),
        _row_spec((RB, HID_F)),
        pl.BlockSpec((1, 1, RB), lambda i: (i, 0, 0)),
        _fix_spec((B, 1)),
    ],
    out_specs=pl.BlockSpec((B, B), lambda i: (0, 0)),
    out_shape=jax.ShapeDtypeStruct((B, B), jnp.float32),
    scratch_shapes=[
        pltpu.VMEM((B, HID_F), jnp.float32),
        pltpu.VMEM((B, HID_F), jnp.float32),
        pltpu.VMEM((B, HID_F), jnp.float32),
    ],
)


@jax.jit
def kernel(x, edge_index, batch, rootindex, W1, b1, W2, b2):
    xp = jnp.pad(x, ((0, NP - N), (0, 0)))
    ei3 = edge_index.reshape(2, EROWS, CW)
    batchp = jnp.concatenate([batch, jnp.full((NP - N,), B, jnp.int32)])
    batch2d = batchp.reshape(NP, 1)
    batchT = batchp.reshape(NBLK, 1, RB)
    ridx2d = rootindex.reshape(B, 1)
    b1t = jnp.tile(b1.reshape(1, HID_F), (8, 1))
    b2t = jnp.tile(b2.reshape(1, HID_F), (8, 1))
    w2a = W2[:HID_F]
    w2b = W2[HID_F:]

    sc_deg = _get_sc_deg()
    sc_edge = _get_sc_edge()
    deg2 = sc_deg(ei3)
    h1, rootsw = _tc1a(xp, W1, ridx2d, w2b)
    u1 = _tc1b(h1, deg2)
    s1 = sc_edge(u1, ei3)
    x2, u2 = _tc2(s1, u1, deg2, b1t, rootsw, batch2d, w2a)
    s2 = sc_edge(u2, ei3)
    out = _tc3(s2, u2, deg2, b2t, x2, batchT, ridx2d)
    return out


# speedup vs baseline: 36.9285x; 1.0017x over previous
import functools

import jax
import jax.numpy as jnp
from jax import lax
from jax.experimental import pallas as pl
from jax.experimental.pallas import tpu as pltpu
from jax.experimental.pallas import tpu_sc as plsc

N = 10000
E = 320000
B = 128
IN_F = 128
HID_F = 64

NP = 10240
NC = 2
NS = 16
NW = NC * NS
CW = 128
EROWS = E // CW
CHUNKS = EROWS // NW
XTRA = EROWS - NW * CHUNKS
ROWS_PER_TILE = NP // NS
RB = 1024
NBLK = NP // RB


def _deg_body(ei_hbm, out_hbm, idx_d, ones_v, sem, acc):
    cid = lax.axis_index("c")
    sid = lax.axis_index("s")
    wid = cid * NS + sid
    nch = CHUNKS + (wid < XTRA).astype(jnp.int32)
    pltpu.sync_copy(ei_hbm.at[1, pl.ds(wid * CHUNKS, CHUNKS)],
                    idx_d.at[pl.ds(0, CHUNKS)])

    @pl.when(wid < XTRA)
    def _():
        pltpu.sync_copy(ei_hbm.at[1, pl.ds(NW * CHUNKS + wid, 1)],
                        idx_d.at[pl.ds(CHUNKS, 1)])

    @pl.loop(0, CW)
    def _zero(r):
        ones_v[r, :] = jnp.zeros((16,), jnp.float32)

    @pl.loop(0, ROWS_PER_TILE // CW)
    def _zcopy(k):
        pltpu.sync_copy(ones_v, acc.at[pl.ds(sid * ROWS_PER_TILE + k * CW, CW)])

    @pl.loop(0, CW)
    def _refill(r):
        ones_v[r, :] = jnp.full((16,), 1.0, jnp.float32)

    plsc.subcore_barrier()

    @pl.loop(0, nch)
    def _scatter(j):
        pltpu.sync_copy(ones_v, acc.at[idx_d.at[j]], add=True)

    plsc.subcore_barrier()
    pltpu.sync_copy(acc.at[pl.ds(sid * ROWS_PER_TILE, ROWS_PER_TILE)],
                    out_hbm.at[cid, pl.ds(sid * ROWS_PER_TILE, ROWS_PER_TILE)])


@functools.cache
def _get_sc_deg():
    mesh = plsc.VectorSubcoreMesh(core_axis_name="c", subcore_axis_name="s")
    return pl.kernel(
        _deg_body,
        out_type=jax.ShapeDtypeStruct((NC, NP, 16), jnp.float32),
        mesh=mesh,
        compiler_params=pltpu.CompilerParams(use_tc_tiling_on_sc=False),
        scratch_types=[
            pltpu.VMEM((CHUNKS + 1, CW), jnp.int32),
            pltpu.VMEM((CW, 16), jnp.float32),
            pltpu.SemaphoreType.DMA,
            pltpu.VMEM_SHARED((NP, 16), jnp.float32),
        ],
    )


def _edge_body(u_hbm, ei_hbm, out_hbm, idx_s, idx_d,
               gsemA, gsemB, ssemA, ssemB, ustage, acc):
    pl.run_scoped(
        functools.partial(_edge_inner, u_hbm, ei_hbm, out_hbm,
                          gsemA, gsemB, ssemA, ssemB, ustage, acc,
                          idx_s, idx_d),
        pltpu.VMEM((2, CW, HID_F), jnp.float32),
    )


def _edge_inner(u_hbm, ei_hbm, out_hbm, gsemA, gsemB, ssemA, ssemB,
                ustage, acc, idx_s, idx_d, rows):
    cid = lax.axis_index("c")
    sid = lax.axis_index("s")
    wid = cid * NS + sid
    nch = CHUNKS + (wid < XTRA).astype(jnp.int32)
    pltpu.sync_copy(ei_hbm.at[0, pl.ds(wid * CHUNKS, CHUNKS)],
                    idx_s.at[pl.ds(0, CHUNKS)])
    pltpu.sync_copy(ei_hbm.at[1, pl.ds(wid * CHUNKS, CHUNKS)],
                    idx_d.at[pl.ds(0, CHUNKS)])

    @pl.when(wid < XTRA)
    def _():
        pltpu.sync_copy(ei_hbm.at[0, pl.ds(NW * CHUNKS + wid, 1)],
                        idx_s.at[pl.ds(CHUNKS, 1)])
        pltpu.sync_copy(ei_hbm.at[1, pl.ds(NW * CHUNKS + wid, 1)],
                        idx_d.at[pl.ds(CHUNKS, 1)])

    pltpu.sync_copy(u_hbm.at[pl.ds(sid * ROWS_PER_TILE, ROWS_PER_TILE)],
                    ustage.at[pl.ds(sid * ROWS_PER_TILE, ROWS_PER_TILE)])

    @pl.loop(0, CW)
    def _zero(r):
        for c in range(HID_F // 16):
            rows[0, r, pl.ds(c * 16, 16)] = jnp.zeros((16,), jnp.float32)

    @pl.loop(0, ROWS_PER_TILE // CW)
    def _zcopy(k):
        pltpu.sync_copy(rows.at[0],
                        acc.at[pl.ds(sid * ROWS_PER_TILE + k * CW, CW)])

    plsc.subcore_barrier()

    pltpu.async_copy(ustage.at[idx_s.at[0]], rows.at[0], gsemA)

    @pl.loop(0, nch)
    def _chunk(j):
        b = jnp.bitwise_and(j, 1)
        pltpu.make_async_copy(ustage.at[idx_s.at[j]], rows.at[b], gsemA).wait()

        @pl.when(j > 0)
        def _():
            pltpu.make_async_copy(rows.at[1 - b], acc.at[idx_d.at[j - 1]],
                                  ssemA).wait()

        @pl.when(j < nch - 1)
        def _():
            pltpu.async_copy(ustage.at[idx_s.at[j + 1]], rows.at[1 - b],
                             gsemA)

        pltpu.async_copy(rows.at[b], acc.at[idx_d.at[j]], ssemA, add=True)

    _lb = jnp.bitwise_and(nch - 1, 1)
    pltpu.make_async_copy(rows.at[_lb], acc.at[idx_d.at[nch - 1]],
                          ssemA).wait()

    plsc.subcore_barrier()
    pltpu.sync_copy(acc.at[pl.ds(sid * ROWS_PER_TILE, ROWS_PER_TILE)],
                    out_hbm.at[cid, pl.ds(sid * ROWS_PER_TILE, ROWS_PER_TILE)])


@functools.cache
def _get_sc_edge():
    mesh = plsc.VectorSubcoreMesh(core_axis_name="c", subcore_axis_name="s")
    return pl.kernel(
        _edge_body,
        out_type=jax.ShapeDtypeStruct((NC, NP, HID_F), jnp.float32),
        mesh=mesh,
        compiler_params=pltpu.CompilerParams(use_tc_tiling_on_sc=False),
        scratch_types=[
            pltpu.VMEM((CHUNKS + 1, CW), jnp.int32),
            pltpu.VMEM((CHUNKS + 1, CW), jnp.int32),
            pltpu.SemaphoreType.DMA,
            pltpu.SemaphoreType.DMA,
            pltpu.SemaphoreType.DMA,
            pltpu.SemaphoreType.DMA,
            pltpu.VMEM_SHARED((NP, HID_F), jnp.float32),
            pltpu.VMEM_SHARED((NP, HID_F), jnp.float32),
        ],
    )


def _dinv_from(deg_ref):
    deg = deg_ref[0, :, 0] + deg_ref[1, :, 0] + 1.0
    return lax.rsqrt(deg)


def _tc1a_body(x_ref, w1_ref, ridx_ref, w2b_ref, h1_ref, rootsw_ref):
    i = pl.program_id(0)
    h1_ref[...] = jnp.dot(x_ref[...], w1_ref[...])
    rows = i * RB + lax.broadcasted_iota(jnp.int32, (1, RB), 1)
    rsel = (ridx_ref[...] == rows).astype(jnp.float32)
    part = jnp.dot(rsel, x_ref[...])
    contrib = jnp.dot(jnp.maximum(part, 0.0), w2b_ref[...])

    @pl.when(i == 0)
    def _():
        rootsw_ref[...] = jnp.zeros_like(rootsw_ref)

    rootsw_ref[...] += contrib


def _tc1b_body(h1_ref, deg_ref, u1_ref):
    dinv = _dinv_from(deg_ref)
    u1_ref[...] = h1_ref[...] * dinv[:, None]


def _tc2_body(s1_ref, u1_ref, deg_ref, b1_ref, rootsw_ref, batch_ref, w2a_ref,
              x2_ref, u2_ref):
    dinv = _dinv_from(deg_ref)
    x2 = (s1_ref[0] + s1_ref[1] + u1_ref[...]) * dinv[:, None] + b1_ref[0:1, :]
    x2_ref[...] = x2
    hr = jnp.maximum(x2, 0.0)
    cols = lax.broadcasted_iota(jnp.int32, (1, B), 1)
    bsel = (batch_ref[:, 0:1] == cols).astype(jnp.float32)
    rext = jnp.dot(bsel, rootsw_ref[...])
    u2_ref[...] = (jnp.dot(hr, w2a_ref[...]) + rext) * dinv[:, None]


def _tc3_body(s2_ref, u2_ref, deg_ref, b2_ref, x2_ref, batchT_ref, ridx_ref,
              out_ref, seg_ref, root_ref, cnt_ref):
    i = pl.program_id(0)

    @pl.when(i == 0)
    def _():
        seg_ref[...] = jnp.zeros_like(seg_ref)
        root_ref[...] = jnp.zeros_like(root_ref)
        cnt_ref[...] = jnp.zeros_like(cnt_ref)

    dinv = _dinv_from(deg_ref)
    g = jnp.maximum((s2_ref[0] + s2_ref[1] + u2_ref[...]) * dinv[:, None]
                    + b2_ref[0:1, :], 0.0)
    biota = lax.broadcasted_iota(jnp.int32, (B, 1), 0)
    bselT = (batchT_ref[0] == biota).astype(jnp.float32)
    seg_ref[...] += jnp.dot(bselT, g)
    ones = jnp.ones((RB, HID_F), jnp.float32)
    cnt_ref[...] += jnp.dot(bselT, ones)
    rows = i * RB + lax.broadcasted_iota(jnp.int32, (1, RB), 1)
    rsel = (ridx_ref[...] == rows).astype(jnp.float32)
    root_ref[...] += jnp.dot(rsel, x2_ref[...])

    @pl.when(i == NBLK - 1)
    def _():
        cnt = cnt_ref[...]
        first = seg_ref[...] / jnp.maximum(cnt, 1.0)
        second = jnp.where(cnt > 0, root_ref[...], 0.0)
        out_ref[...] = jnp.concatenate([first, second], axis=1)


def _row_spec(shape):
    return pl.BlockSpec(shape, lambda i: (i, 0))


def _fix_spec(shape):
    return pl.BlockSpec(shape, lambda i: (0, 0))


_DEG_SPEC = pl.BlockSpec((NC, RB, 16), lambda i: (0, i, 0))
_PAIR_SPEC = pl.BlockSpec((NC, RB, HID_F), lambda i: (0, i, 0))
_U_SPEC = pl.BlockSpec((RB, HID_F), lambda i: (i, 0))

_tc1a = pl.pallas_call(
    _tc1a_body,
    grid=(NBLK,),
    in_specs=[
        _row_spec((RB, IN_F)),
        _fix_spec((IN_F, HID_F)),
        _fix_spec((B, 1)),
        _fix_spec((IN_F, HID_F)),
    ],
    out_specs=[_row_spec((RB, HID_F)), _fix_spec((B, HID_F))],
    out_shape=[
        jax.ShapeDtypeStruct((NP, HID_F), jnp.float32),
        jax.ShapeDtypeStruct((B, HID_F), jnp.float32),
    ],
)

_tc1b = pl.pallas_call(
    _tc1b_body,
    grid=(NBLK,),
    in_specs=[_row_spec((RB, HID_F)), _DEG_SPEC],
    out_specs=_U_SPEC,
    out_shape=jax.ShapeDtypeStruct((NP, HID_F), jnp.float32),
)

_tc2 = pl.pallas_call(
    _tc2_body,
    grid=(NBLK,),
    in_specs=[
        _PAIR_SPEC,
        _U_SPEC,
        _DEG_SPEC,
        _fix_spec((8, HID_F)),
        _fix_spec((B, HID_F)),
        _row_spec((RB, 1)),
        _fix_spec((HID_F, HID_F)),
    ],
    out_specs=[_row_spec((RB, HID_F)), _U_SPEC],
    out_shape=[
        jax.ShapeDtypeStruct((NP, HID_F), jnp.float32),
        jax.ShapeDtypeStruct((NP, HID_F), jnp.float32),
    ],
)

_tc3 = pl.pallas_call(
    _tc3_body,
    grid=(NBLK,),
    in_specs=[
        _PAIR_SPEC,
        _U_SPEC,
        _DEG_SPEC,
        _fix_spec((8, HID_F)),
        _row_spec((RB, HID_F)),
        pl.BlockSpec((1, 1, RB), lambda i: (i, 0, 0)),
        _fix_spec((B, 1)),
    ],
    out_specs=pl.BlockSpec((B, B), lambda i: (0, 0)),
    out_shape=jax.ShapeDtypeStruct((B, B), jnp.float32),
    scratch_shapes=[
        pltpu.VMEM((B, HID_F), jnp.float32),
        pltpu.VMEM((B, HID_F), jnp.float32),
        pltpu.VMEM((B, HID_F), jnp.float32),
    ],
)


@jax.jit
def kernel(x, edge_index, batch, rootindex, W1, b1, W2, b2):
    xp = jnp.pad(x, ((0, NP - N), (0, 0)))
    ei3 = edge_index.reshape(2, EROWS, CW)
    batchp = jnp.concatenate([batch, jnp.full((NP - N,), B, jnp.int32)])
    batch2d = batchp.reshape(NP, 1)
    batchT = batchp.reshape(NBLK, 1, RB)
    ridx2d = rootindex.reshape(B, 1)
    b1t = jnp.tile(b1.reshape(1, HID_F), (8, 1))
    b2t = jnp.tile(b2.reshape(1, HID_F), (8, 1))
    w2a = W2[:HID_F]
    w2b = W2[HID_F:]

    sc_deg = _get_sc_deg()
    sc_edge = _get_sc_edge()
    deg2 = sc_deg(ei3)
    h1, rootsw = _tc1a(xp, W1, ridx2d, w2b)
    u1 = _tc1b(h1, deg2)
    s1 = sc_edge(u1, ei3)
    x2, u2 = _tc2(s1, u1, deg2, b1t, rootsw, batch2d, w2a)
    s2 = sc_edge(u2, ei3)
    out = _tc3(s2, u2, deg2, b2t, x2, batchT, ridx2d)
    return out
